# baseline (speedup 1.0000x reference)
import functools

import jax
import jax.numpy as jnp
from jax import lax
from jax.experimental import pallas as pl
from jax.experimental.pallas import tpu as pltpu
from jax.experimental.pallas import tpu_sc as plsc

E = 8
AUX_COEF = 0.01
NEG = -1e30
BMD = 1024
SB = 512
NBP = 12
HT = 512
TP = NBP * BMD


def _shift_right(a, d):
    pad = jnp.zeros((a.shape[0], d), a.dtype)
    return jnp.concatenate([pad, a[:, :-d]], axis=1)


def _cumsum_lanes(a):
    d = 1
    while d < a.shape[1]:
        a = a + _shift_right(a, d)
        d *= 2
    return a


def _router_body(x_ref, rw_ref, aux_ref, pos1_ref, pos2_ref,
                 wb1_ref, wb2_ref, sinfo_ref):
    x = x_ref[...]
    rw = rw_ref[...]
    logits = jax.lax.dot_general(
        x, rw, (((1,), (1,)), ((), ())),
        preferred_element_type=jnp.float32)
    T = logits.shape[0]
    col = jax.lax.broadcasted_iota(jnp.int32, logits.shape, 1)
    valid = col < E
    logits = jnp.where(valid, logits, NEG)
    m = jnp.max(logits, axis=1, keepdims=True)
    ex = jnp.where(valid, jnp.exp(logits - m), 0.0)
    probs = ex / jnp.sum(ex, axis=1, keepdims=True)
    m1 = jnp.max(logits, axis=1, keepdims=True)
    am1 = jnp.min(jnp.where(logits == m1, col, 9999), axis=1, keepdims=True)
    p1 = jnp.max(probs, axis=1, keepdims=True)
    logits2 = jnp.where(col == am1, NEG, logits)
    m2 = jnp.max(logits2, axis=1, keepdims=True)
    am2 = jnp.min(jnp.where(logits2 == m2, col, 9999), axis=1, keepdims=True)
    p2 = jnp.max(jnp.where(col == am1, -1.0, probs), axis=1, keepdims=True)
    one1 = (col == am1).astype(jnp.float32)
    dens = jnp.sum(one1, axis=0, keepdims=True) / T
    rpm = jnp.sum(probs, axis=0, keepdims=True) / T
    aux_ref[...] = AUX_COEF * E * jnp.sum(dens * rpm, keepdims=True)
    wb1_ref[...] = jnp.broadcast_to(p1, (T, 128))
    wb2_ref[...] = jnp.broadcast_to(p2, (T, 128))
    ri = jax.lax.broadcasted_iota(jnp.int32, (T, T), 0)
    rj = jax.lax.broadcasted_iota(jnp.int32, (T, T), 1)
    lt = (ri >= rj).astype(jnp.float32)
    oh1f = (col == am1).astype(jnp.float32)
    oh2f = (col == am2).astype(jnp.float32)
    oh1 = oh1f.astype(jnp.int32)
    oh2 = oh2f.astype(jnp.int32)
    cum1 = jax.lax.dot_general(lt, oh1f, (((1,), (0,)), ((), ())),
                               preferred_element_type=jnp.float32).astype(jnp.int32)
    cum2 = jax.lax.dot_general(lt, oh2f, (((1,), (0,)), ((), ())),
                               preferred_element_type=jnp.float32).astype(jnp.int32)
    cnt1 = cum1[-1:, :]
    cnt2 = cum2[-1:, :]
    counts = cnt1 + cnt2
    nblk = (counts + (BMD - 1)) // BMD
    cumblk = _cumsum_lanes(nblk)
    excl = cumblk - nblk
    nu = jnp.sum(nblk, axis=1, keepdims=True)
    start_pad = excl * BMD
    pos1 = jnp.sum(oh1 * (start_pad + cum1 - 1), axis=1, keepdims=True)
    pos2 = jnp.sum(oh2 * (start_pad + cnt1 + cum2 - 1), axis=1, keepdims=True)
    pos1_ref[...] = pos1
    pos2_ref[...] = pos2
    biota = jax.lax.broadcasted_iota(jnp.int32, (1, 128), 1)
    mk = jnp.zeros((1, 128), jnp.int32)
    prev = jnp.zeros((1, 1), jnp.int32)
    for e in range(E):
        has = nblk[:, e:e + 1] > 0
        delta = jnp.where(has, e - prev, 0)
        mk = mk + jnp.where(biota == excl[:, e:e + 1], delta, 0)
        prev = jnp.where(has, e, prev)
    sblk = _cumsum_lanes(mk)
    exclb = jnp.zeros((1, 128), jnp.int32)
    cb = jnp.zeros((1, 128), jnp.int32)
    for e in range(E):
        sel = (sblk == e).astype(jnp.int32)
        exclb = exclb + sel * excl[:, e:e + 1]
        cb = cb + sel * counts[:, e:e + 1]
    vcnt = jnp.clip(cb - (biota - exclb) * BMD, 0, BMD)
    info = jnp.where(biota == 16, jnp.broadcast_to(nu, (1, 128)), sblk)
    info = jnp.where((biota >= 32) & (biota < 48), _shift_right(vcnt, 32), info)
    sinfo_ref[...] = info


def _make_router(T, D):
    return pl.pallas_call(
        _router_body,
        out_shape=[
            jax.ShapeDtypeStruct((1, 1), jnp.float32),
            jax.ShapeDtypeStruct((T, 1), jnp.int32),
            jax.ShapeDtypeStruct((T, 1), jnp.int32),
            jax.ShapeDtypeStruct((T, 128), jnp.float32),
            jax.ShapeDtypeStruct((T, 128), jnp.float32),
            jax.ShapeDtypeStruct((1, 128), jnp.int32),
        ],
    )


def _make_dispatch(T, D):
    info = plsc.get_sparse_core_info()
    nw = info.num_cores * info.num_subcores
    nc = info.num_cores
    ch = T // nw
    mesh = plsc.VectorSubcoreMesh(core_axis_name="c", subcore_axis_name="s")

    @functools.partial(
        pl.kernel, mesh=mesh,
        out_type=[
            jax.ShapeDtypeStruct((TP, D), jnp.float32),
            jax.ShapeDtypeStruct((TP, 128), jnp.float32),
        ],
        scratch_types=[
            pltpu.VMEM((ch, D), jnp.float32),
            pltpu.VMEM((ch,), jnp.int32),
            pltpu.VMEM((ch,), jnp.int32),
            pltpu.VMEM((ch, 128), jnp.float32),
            pltpu.VMEM((ch, 128), jnp.float32),
            pltpu.SemaphoreType.DMA,
        ],
    )
    def dispatch(xf, pos1, pos2, wb1, wb2, xs_out, wi_out,
                 xb, idx1, idx2, ib1, ib2, sem):
        wid = lax.axis_index("s") * nc + lax.axis_index("c")
        base = wid * ch
        pltpu.sync_copy(xf.at[pl.ds(base, ch)], xb)
        pltpu.sync_copy(pos1.at[pl.ds(base, ch)], idx1)
        pltpu.sync_copy(pos2.at[pl.ds(base, ch)], idx2)
        pltpu.sync_copy(wb1.at[pl.ds(base, ch)], ib1)
        pltpu.sync_copy(wb2.at[pl.ds(base, ch)], ib2)
        c1 = pltpu.async_copy(xb, xs_out.at[idx1], sem)
        c2 = pltpu.async_copy(xb, xs_out.at[idx2], sem)
        c3 = pltpu.async_copy(ib1, wi_out.at[idx1], sem)
        c4 = pltpu.async_copy(ib2, wi_out.at[idx2], sem)
        c1.wait()
        c2.wait()
        c3.wait()
        c4.wait()

    return dispatch


def _ffn_body(sblk_ref, snu_ref, svc_ref, xs_ref, w1_ref, w3_ref, w2_ref,
              wi_ref, ys_ref):
    b = pl.program_id(0)
    h = pl.program_id(1)
    vc = svc_ref[b]
    w1 = w1_ref[0]
    w3 = w3_ref[0]
    w2 = w2_ref[0]

    def sub(s):
        sl = pl.ds(s * SB, SB)
        x = xs_ref[sl, :]
        wc = wi_ref[sl, 0:1]
        hh = jax.lax.dot_general(x, w1, (((1,), (1,)), ((), ())),
                                 preferred_element_type=jnp.float32)
        g = hh * jax.nn.sigmoid(hh)
        u = jax.lax.dot_general(x, w3, (((1,), (1,)), ((), ())),
                                preferred_element_type=jnp.float32)
        act = (g * u) * wc
        y = jax.lax.dot_general(act, w2, (((1,), (1,)), ((), ())),
                                preferred_element_type=jnp.float32)

        @pl.when(h == 0)
        def _():
            ys_ref[sl, :] = y

        @pl.when(h != 0)
        def _():
            ys_ref[sl, :] += y

    @pl.when(vc > 0)
    def _():
        sub(0)

    @pl.when(vc > SB)
    def _():
        sub(1)


def _make_ffn(D, H):
    nh = H // HT

    def xs_idx(b, h, sblk, snu, svc):
        return (jnp.minimum(b, snu[0] - 1), 0)

    def w13_idx(b, h, sblk, snu, svc):
        return (sblk[b], jnp.where(b < snu[0], h, nh - 1), 0)

    def w2_idx(b, h, sblk, snu, svc):
        return (sblk[b], 0, jnp.where(b < snu[0], h, nh - 1))

    grid_spec = pltpu.PrefetchScalarGridSpec(
        num_scalar_prefetch=3,
        grid=(NBP, nh),
        in_specs=[
            pl.BlockSpec((BMD, D), xs_idx),
            pl.BlockSpec((1, HT, D), w13_idx),
            pl.BlockSpec((1, HT, D), w13_idx),
            pl.BlockSpec((1, D, HT), w2_idx),
            pl.BlockSpec((BMD, 128), xs_idx),
        ],
        out_specs=pl.BlockSpec((BMD, D), xs_idx),
    )
    return pl.pallas_call(
        _ffn_body,
        grid_spec=grid_spec,
        out_shape=jax.ShapeDtypeStruct((TP, D), jnp.float32),
        compiler_params=pltpu.CompilerParams(
            dimension_semantics=("arbitrary", "arbitrary")),
    )


def _make_combine(T, D):
    info = plsc.get_sparse_core_info()
    nw = info.num_cores * info.num_subcores
    nc = info.num_cores
    per_w = T // nw
    ch = 16

    mesh = plsc.VectorSubcoreMesh(core_axis_name="c", subcore_axis_name="s")

    @functools.partial(
        pl.kernel, mesh=mesh,
        out_type=[
            jax.ShapeDtypeStruct((T, D), jnp.float32),
            jax.ShapeDtypeStruct((T, D), jnp.float32),
        ],
        scratch_types=[
            pltpu.VMEM((ch,), jnp.int32),
            pltpu.VMEM((ch,), jnp.int32),
            pltpu.VMEM((ch, D), jnp.float32),
            pltpu.VMEM((ch, D), jnp.float32),
            pltpu.SemaphoreType.DMA,
        ],
    )
    def combine(ys, pos1, pos2, y1_out, y2_out, idx1, idx2, r1, r2, sem):
        wid = lax.axis_index("s") * nc + lax.axis_index("c")
        base = wid * per_w

        def step(c, carry):
            off = base + c * ch
            pltpu.sync_copy(pos1.at[pl.ds(off, ch)], idx1)
            pltpu.sync_copy(pos2.at[pl.ds(off, ch)], idx2)
            g1 = pltpu.async_copy(ys.at[idx1], r1, sem)
            g2 = pltpu.async_copy(ys.at[idx2], r2, sem)
            g1.wait()
            g2.wait()
            pltpu.sync_copy(r1, y1_out.at[pl.ds(off, ch)])
            pltpu.sync_copy(r2, y2_out.at[pl.ds(off, ch)])
            return carry

        lax.fori_loop(0, per_w // ch, step, 0)

    return combine


def _add_body(y1_ref, y2_ref, out_ref):
    out_ref[...] = y1_ref[...] + y2_ref[...]


def _make_add(T, D):
    bt = 512
    return pl.pallas_call(
        _add_body,
        grid=(T // bt,),
        in_specs=[
            pl.BlockSpec((bt, D), lambda i: (i, 0)),
            pl.BlockSpec((bt, D), lambda i: (i, 0)),
        ],
        out_specs=pl.BlockSpec((bt, D), lambda i: (i, 0)),
        out_shape=jax.ShapeDtypeStruct((T, D), jnp.float32),
    )


def kernel(x, router_w, w1, w2, w3):
    B, S, D = x.shape
    T = B * S
    H = w1.shape[1]
    xf = x.reshape(T, D)
    rw_pad = jnp.zeros((128, D), jnp.float32).at[:E].set(router_w)

    aux, pos1, pos2, wb1, wb2, sinfo = _make_router(T, D)(xf, rw_pad)
    sblk = sinfo[0, :NBP]
    snu = sinfo[0, 16:17]
    svc = sinfo[0, 32:32 + NBP]
    pos1f = pos1.reshape(T)
    pos2f = pos2.reshape(T)

    xs, winfo = _make_dispatch(T, D)(xf, pos1f, pos2f, wb1, wb2)
    ys = _make_ffn(D, H)(sblk, snu, svc, xs, w1, w3, w2, winfo)
    y1, y2 = _make_combine(T, D)(ys, pos1f, pos2f)
    out = _make_add(T, D)(y1, y2)
    return out.reshape(B, S, D), aux[0, 0]


# speedup vs baseline: 1.0924x; 1.0924x over previous
import functools

import jax
import jax.numpy as jnp
from jax import lax
from jax.experimental import pallas as pl
from jax.experimental.pallas import tpu as pltpu
from jax.experimental.pallas import tpu_sc as plsc

E = 8
AUX_COEF = 0.01
NEG = -1e30
BM = 512
NB = 16
HT = 512
TP = NB * BM


def _shift_down(a, d):
    pad = jnp.zeros((d, a.shape[1]), a.dtype)
    return jnp.concatenate([pad, a[:-d]], axis=0)


def _shift_right(a, d):
    pad = jnp.zeros((a.shape[0], d), a.dtype)
    return jnp.concatenate([pad, a[:, :-d]], axis=1)


def _cumsum_rows(a):
    d = 1
    while d < a.shape[0]:
        a = a + _shift_down(a, d)
        d *= 2
    return a


def _cumsum_lanes(a):
    d = 1
    while d < a.shape[1]:
        a = a + _shift_right(a, d)
        d *= 2
    return a


def _router_body(x_ref, rw_ref, aux_ref, pos1_ref, pos2_ref,
                 p1_ref, p2_ref, sinfo_ref):
    x = x_ref[...]
    rw = rw_ref[...]
    logits = jax.lax.dot_general(
        x, rw, (((1,), (1,)), ((), ())),
        preferred_element_type=jnp.float32)
    T = logits.shape[0]
    col = jax.lax.broadcasted_iota(jnp.int32, logits.shape, 1)
    valid = col < E
    logits = jnp.where(valid, logits, NEG)
    m = jnp.max(logits, axis=1, keepdims=True)
    ex = jnp.where(valid, jnp.exp(logits - m), 0.0)
    probs = ex / jnp.sum(ex, axis=1, keepdims=True)
    m1 = jnp.max(logits, axis=1, keepdims=True)
    am1 = jnp.min(jnp.where(logits == m1, col, 9999), axis=1, keepdims=True)
    p1 = jnp.max(probs, axis=1, keepdims=True)
    logits2 = jnp.where(col == am1, NEG, logits)
    m2 = jnp.max(logits2, axis=1, keepdims=True)
    am2 = jnp.min(jnp.where(logits2 == m2, col, 9999), axis=1, keepdims=True)
    p2 = jnp.max(jnp.where(col == am1, -1.0, probs), axis=1, keepdims=True)
    one1 = (col == am1).astype(jnp.float32)
    dens = jnp.sum(one1, axis=0, keepdims=True) / T
    rpm = jnp.sum(probs, axis=0, keepdims=True) / T
    aux_ref[...] = AUX_COEF * E * jnp.sum(dens * rpm, keepdims=True)
    p1_ref[...] = p1
    p2_ref[...] = p2
    oh1 = (col == am1).astype(jnp.int32)
    oh2 = (col == am2).astype(jnp.int32)
    cum1 = _cumsum_rows(oh1)
    cum2 = _cumsum_rows(oh2)
    cnt1 = cum1[-1:, :]
    cnt2 = cum2[-1:, :]
    counts = cnt1 + cnt2
    nblk = (counts + (BM - 1)) // BM
    cumblk = _cumsum_lanes(nblk)
    excl = cumblk - nblk
    nu = jnp.sum(nblk, axis=1, keepdims=True)
    start_pad = excl * BM
    pos1 = jnp.sum(oh1 * (start_pad + cum1 - 1), axis=1, keepdims=True)
    pos2 = jnp.sum(oh2 * (start_pad + cnt1 + cum2 - 1), axis=1, keepdims=True)
    pos1_ref[...] = pos1
    pos2_ref[...] = pos2
    biota = jax.lax.broadcasted_iota(jnp.int32, (1, 128), 1)
    mk = jnp.zeros((1, 128), jnp.int32)
    prev = jnp.zeros((1, 1), jnp.int32)
    for e in range(E):
        has = nblk[:, e:e + 1] > 0
        delta = jnp.where(has, e - prev, 0)
        mk = mk + jnp.where(biota == excl[:, e:e + 1], delta, 0)
        prev = jnp.where(has, e, prev)
    sblk = _cumsum_lanes(mk)
    sinfo_ref[...] = jnp.where(biota == 16, jnp.broadcast_to(nu, (1, 128)), sblk)


def _make_router(T, D):
    return pl.pallas_call(
        _router_body,
        out_shape=[
            jax.ShapeDtypeStruct((1, 1), jnp.float32),
            jax.ShapeDtypeStruct((T, 1), jnp.int32),
            jax.ShapeDtypeStruct((T, 1), jnp.int32),
            jax.ShapeDtypeStruct((T, 1), jnp.float32),
            jax.ShapeDtypeStruct((T, 1), jnp.float32),
            jax.ShapeDtypeStruct((1, 128), jnp.int32),
        ],
    )


def _make_dispatch(T, D):
    info = plsc.get_sparse_core_info()
    nw = info.num_cores * info.num_subcores
    nc = info.num_cores
    ch = T // nw
    mesh = plsc.VectorSubcoreMesh(core_axis_name="c", subcore_axis_name="s")

    @functools.partial(
        pl.kernel, mesh=mesh,
        out_type=jax.ShapeDtypeStruct((TP, D), jnp.float32),
        scratch_types=[
            pltpu.VMEM((ch, D), jnp.float32),
            pltpu.VMEM((ch,), jnp.int32),
            pltpu.VMEM((ch,), jnp.int32),
            pltpu.SemaphoreType.DMA,
        ],
    )
    def dispatch(xf, pos1, pos2, xs_out, xb, idx1, idx2, sem):
        wid = lax.axis_index("s") * nc + lax.axis_index("c")
        base = wid * ch
        pltpu.sync_copy(xf.at[pl.ds(base, ch)], xb)
        pltpu.sync_copy(pos1.at[pl.ds(base, ch)], idx1)
        pltpu.sync_copy(pos2.at[pl.ds(base, ch)], idx2)
        c1 = pltpu.async_copy(xb, xs_out.at[idx1], sem)
        c2 = pltpu.async_copy(xb, xs_out.at[idx2], sem)
        c1.wait()
        c2.wait()

    return dispatch


def _ffn_body(sblk_ref, snu_ref, xs_ref, w1_ref, w3_ref, w2_ref, ys_ref):
    b = pl.program_id(0)
    h = pl.program_id(1)

    @pl.when(b < snu_ref[0])
    def _():
        x = xs_ref[...]
        w1 = w1_ref[0]
        w3 = w3_ref[0]
        w2 = w2_ref[0]
        hh = jax.lax.dot_general(x, w1, (((1,), (1,)), ((), ())),
                                 preferred_element_type=jnp.float32)
        g = hh * jax.nn.sigmoid(hh)
        u = jax.lax.dot_general(x, w3, (((1,), (1,)), ((), ())),
                                preferred_element_type=jnp.float32)
        act = g * u
        y = jax.lax.dot_general(act, w2, (((1,), (1,)), ((), ())),
                                preferred_element_type=jnp.float32)

        @pl.when(h == 0)
        def _():
            ys_ref[...] = y

        @pl.when(h != 0)
        def _():
            ys_ref[...] += y


def _make_ffn(D, H):
    nh = H // HT

    def xs_idx(b, h, sblk, snu):
        return (jnp.minimum(b, snu[0] - 1), 0)

    def w13_idx(b, h, sblk, snu):
        return (sblk[b], jnp.where(b < snu[0], h, nh - 1), 0)

    def w2_idx(b, h, sblk, snu):
        return (sblk[b], 0, jnp.where(b < snu[0], h, nh - 1))

    grid_spec = pltpu.PrefetchScalarGridSpec(
        num_scalar_prefetch=2,
        grid=(NB, nh),
        in_specs=[
            pl.BlockSpec((BM, D), xs_idx),
            pl.BlockSpec((1, HT, D), w13_idx),
            pl.BlockSpec((1, HT, D), w13_idx),
            pl.BlockSpec((1, D, HT), w2_idx),
        ],
        out_specs=pl.BlockSpec((BM, D), xs_idx),
    )
    return pl.pallas_call(
        _ffn_body,
        grid_spec=grid_spec,
        out_shape=jax.ShapeDtypeStruct((TP, D), jnp.float32),
        compiler_params=pltpu.CompilerParams(
            dimension_semantics=("arbitrary", "arbitrary")),
    )


def _make_combine(T, D):
    info = plsc.get_sparse_core_info()
    nw = info.num_cores * info.num_subcores
    nc = info.num_cores
    per_w = T // nw
    ch = 16

    mesh = plsc.VectorSubcoreMesh(core_axis_name="c", subcore_axis_name="s")

    @functools.partial(
        pl.kernel, mesh=mesh,
        out_type=[
            jax.ShapeDtypeStruct((T, D), jnp.float32),
            jax.ShapeDtypeStruct((T, D), jnp.float32),
        ],
        scratch_types=[
            pltpu.VMEM((ch,), jnp.int32),
            pltpu.VMEM((ch,), jnp.int32),
            pltpu.VMEM((ch, D), jnp.float32),
            pltpu.VMEM((ch, D), jnp.float32),
            pltpu.SemaphoreType.DMA,
        ],
    )
    def combine(ys, pos1, pos2, y1_out, y2_out, idx1, idx2, r1, r2, sem):
        wid = lax.axis_index("s") * nc + lax.axis_index("c")
        base = wid * per_w

        def step(c, carry):
            off = base + c * ch
            pltpu.sync_copy(pos1.at[pl.ds(off, ch)], idx1)
            pltpu.sync_copy(pos2.at[pl.ds(off, ch)], idx2)
            g1 = pltpu.async_copy(ys.at[idx1], r1, sem)
            g2 = pltpu.async_copy(ys.at[idx2], r2, sem)
            g1.wait()
            g2.wait()
            pltpu.sync_copy(r1, y1_out.at[pl.ds(off, ch)])
            pltpu.sync_copy(r2, y2_out.at[pl.ds(off, ch)])
            return carry

        lax.fori_loop(0, per_w // ch, step, 0)

    return combine


def _wsum_body(y1_ref, y2_ref, p1_ref, p2_ref, out_ref):
    out_ref[...] = p1_ref[...] * y1_ref[...] + p2_ref[...] * y2_ref[...]


def _make_wsum(T, D):
    bt = 512
    return pl.pallas_call(
        _wsum_body,
        grid=(T // bt,),
        in_specs=[
            pl.BlockSpec((bt, D), lambda i: (i, 0)),
            pl.BlockSpec((bt, D), lambda i: (i, 0)),
            pl.BlockSpec((bt, 1), lambda i: (i, 0)),
            pl.BlockSpec((bt, 1), lambda i: (i, 0)),
        ],
        out_specs=pl.BlockSpec((bt, D), lambda i: (i, 0)),
        out_shape=jax.ShapeDtypeStruct((T, D), jnp.float32),
    )


def kernel(x, router_w, w1, w2, w3):
    B, S, D = x.shape
    T = B * S
    H = w1.shape[1]
    xf = x.reshape(T, D)
    rw_pad = jnp.zeros((128, D), jnp.float32).at[:E].set(router_w)

    aux, pos1, pos2, p1, p2, sinfo = _make_router(T, D)(xf, rw_pad)
    sblk = sinfo[0, :NB]
    snu = sinfo[0, 16:17]
    pos1f = pos1.reshape(T)
    pos2f = pos2.reshape(T)

    xs = _make_dispatch(T, D)(xf, pos1f, pos2f)
    ys = _make_ffn(D, H)(sblk, snu, xs, w1, w3, w2)
    y1, y2 = _make_combine(T, D)(ys, pos1f, pos2f)
    out = _make_wsum(T, D)(y1, y2, p1, p2)
    return out.reshape(B, S, D), aux[0, 0]


# speedup vs baseline: 1.2268x; 1.1231x over previous
import functools

import jax
import jax.numpy as jnp
from jax import lax
from jax.experimental import pallas as pl
from jax.experimental.pallas import tpu as pltpu
from jax.experimental.pallas import tpu_sc as plsc

E = 8
AUX_COEF = 0.01
NEG = -1e30
BM = 512
NB = 16
HT = 1024
TP = NB * BM


def _shift_down(a, d):
    pad = jnp.zeros((d, a.shape[1]), a.dtype)
    return jnp.concatenate([pad, a[:-d]], axis=0)


def _shift_right(a, d):
    pad = jnp.zeros((a.shape[0], d), a.dtype)
    return jnp.concatenate([pad, a[:, :-d]], axis=1)


def _cumsum_rows(a):
    d = 1
    while d < a.shape[0]:
        a = a + _shift_down(a, d)
        d *= 2
    return a


def _cumsum_lanes(a):
    d = 1
    while d < a.shape[1]:
        a = a + _shift_right(a, d)
        d *= 2
    return a


def _router_body(x_ref, rw_ref, aux_ref, pos1_ref, pos2_ref,
                 p1_ref, p2_ref, sinfo_ref):
    x = x_ref[...]
    rw = rw_ref[...]
    logits = jax.lax.dot_general(
        x, rw, (((1,), (1,)), ((), ())),
        preferred_element_type=jnp.float32)
    T = logits.shape[0]
    col = jax.lax.broadcasted_iota(jnp.int32, logits.shape, 1)
    valid = col < E
    logits = jnp.where(valid, logits, NEG)
    m = jnp.max(logits, axis=1, keepdims=True)
    ex = jnp.where(valid, jnp.exp(logits - m), 0.0)
    probs = ex / jnp.sum(ex, axis=1, keepdims=True)
    m1 = jnp.max(logits, axis=1, keepdims=True)
    am1 = jnp.min(jnp.where(logits == m1, col, 9999), axis=1, keepdims=True)
    p1 = jnp.max(probs, axis=1, keepdims=True)
    logits2 = jnp.where(col == am1, NEG, logits)
    m2 = jnp.max(logits2, axis=1, keepdims=True)
    am2 = jnp.min(jnp.where(logits2 == m2, col, 9999), axis=1, keepdims=True)
    p2 = jnp.max(jnp.where(col == am1, -1.0, probs), axis=1, keepdims=True)
    one1 = (col == am1).astype(jnp.float32)
    dens = jnp.sum(one1, axis=0, keepdims=True) / T
    rpm = jnp.sum(probs, axis=0, keepdims=True) / T
    aux_ref[...] = AUX_COEF * E * jnp.sum(dens * rpm, keepdims=True)
    p1_ref[...] = p1
    p2_ref[...] = p2
    oh1 = (col == am1).astype(jnp.int32)
    oh2 = (col == am2).astype(jnp.int32)
    cum1 = _cumsum_rows(oh1)
    cum2 = _cumsum_rows(oh2)
    cnt1 = cum1[-1:, :]
    cnt2 = cum2[-1:, :]
    counts = cnt1 + cnt2
    nblk = (counts + (BM - 1)) // BM
    cumblk = _cumsum_lanes(nblk)
    excl = cumblk - nblk
    nu = jnp.sum(nblk, axis=1, keepdims=True)
    start_pad = excl * BM
    pos1 = jnp.sum(oh1 * (start_pad + cum1 - 1), axis=1, keepdims=True)
    pos2 = jnp.sum(oh2 * (start_pad + cnt1 + cum2 - 1), axis=1, keepdims=True)
    pos1_ref[...] = pos1
    pos2_ref[...] = pos2
    biota = jax.lax.broadcasted_iota(jnp.int32, (1, 128), 1)
    mk = jnp.zeros((1, 128), jnp.int32)
    prev = jnp.zeros((1, 1), jnp.int32)
    for e in range(E):
        has = nblk[:, e:e + 1] > 0
        delta = jnp.where(has, e - prev, 0)
        mk = mk + jnp.where(biota == excl[:, e:e + 1], delta, 0)
        prev = jnp.where(has, e, prev)
    sblk = _cumsum_lanes(mk)
    sinfo_ref[...] = jnp.where(biota == 16, jnp.broadcast_to(nu, (1, 128)), sblk)


def _make_router(T, D):
    return pl.pallas_call(
        _router_body,
        out_shape=[
            jax.ShapeDtypeStruct((1, 1), jnp.float32),
            jax.ShapeDtypeStruct((T, 1), jnp.int32),
            jax.ShapeDtypeStruct((T, 1), jnp.int32),
            jax.ShapeDtypeStruct((T, 1), jnp.float32),
            jax.ShapeDtypeStruct((T, 1), jnp.float32),
            jax.ShapeDtypeStruct((1, 128), jnp.int32),
        ],
    )


def _make_dispatch(T, D):
    info = plsc.get_sparse_core_info()
    nw = info.num_cores * info.num_subcores
    nc = info.num_cores
    ch = T // nw
    mesh = plsc.VectorSubcoreMesh(core_axis_name="c", subcore_axis_name="s")

    @functools.partial(
        pl.kernel, mesh=mesh,
        out_type=jax.ShapeDtypeStruct((TP, D), jnp.float32),
        scratch_types=[
            pltpu.VMEM((ch, D), jnp.float32),
            pltpu.VMEM((ch,), jnp.int32),
            pltpu.VMEM((ch,), jnp.int32),
            pltpu.SemaphoreType.DMA,
        ],
    )
    def dispatch(xf, pos1, pos2, xs_out, xb, idx1, idx2, sem):
        wid = lax.axis_index("s") * nc + lax.axis_index("c")
        base = wid * ch
        pltpu.sync_copy(xf.at[pl.ds(base, ch)], xb)
        pltpu.sync_copy(pos1.at[pl.ds(base, ch)], idx1)
        pltpu.sync_copy(pos2.at[pl.ds(base, ch)], idx2)
        c1 = pltpu.async_copy(xb, xs_out.at[idx1], sem)
        c2 = pltpu.async_copy(xb, xs_out.at[idx2], sem)
        c1.wait()
        c2.wait()

    return dispatch


def _ffn_body(sblk_ref, snu_ref, xs_ref, w1_ref, w3_ref, w2_ref, ys_ref):
    b = pl.program_id(0)
    h = pl.program_id(1)

    @pl.when(b < snu_ref[0])
    def _():
        x = xs_ref[...]
        w1 = w1_ref[0]
        w3 = w3_ref[0]
        w2 = w2_ref[0]
        hh = jax.lax.dot_general(x, w1, (((1,), (1,)), ((), ())),
                                 preferred_element_type=jnp.float32)
        g = hh * jax.nn.sigmoid(hh)
        u = jax.lax.dot_general(x, w3, (((1,), (1,)), ((), ())),
                                preferred_element_type=jnp.float32)
        act = g * u
        y = jax.lax.dot_general(act, w2, (((1,), (1,)), ((), ())),
                                preferred_element_type=jnp.float32)

        @pl.when(h == 0)
        def _():
            ys_ref[...] = y

        @pl.when(h != 0)
        def _():
            ys_ref[...] += y


def _make_ffn(D, H):
    nh = H // HT

    def xs_idx(b, h, sblk, snu):
        return (jnp.minimum(b, snu[0] - 1), 0)

    def w13_idx(b, h, sblk, snu):
        return (sblk[b], jnp.where(b < snu[0], h, nh - 1), 0)

    def w2_idx(b, h, sblk, snu):
        return (sblk[b], 0, jnp.where(b < snu[0], h, nh - 1))

    grid_spec = pltpu.PrefetchScalarGridSpec(
        num_scalar_prefetch=2,
        grid=(NB, nh),
        in_specs=[
            pl.BlockSpec((BM, D), xs_idx),
            pl.BlockSpec((1, HT, D), w13_idx),
            pl.BlockSpec((1, HT, D), w13_idx),
            pl.BlockSpec((1, D, HT), w2_idx),
        ],
        out_specs=pl.BlockSpec((BM, D), xs_idx),
    )
    return pl.pallas_call(
        _ffn_body,
        grid_spec=grid_spec,
        out_shape=jax.ShapeDtypeStruct((TP, D), jnp.float32),
        compiler_params=pltpu.CompilerParams(
            dimension_semantics=("arbitrary", "arbitrary")),
    )


def _make_combine(T, D):
    info = plsc.get_sparse_core_info()
    nw = info.num_cores * info.num_subcores
    nc = info.num_cores
    per_w = T // nw
    ch = 16

    mesh = plsc.VectorSubcoreMesh(core_axis_name="c", subcore_axis_name="s")

    @functools.partial(
        pl.kernel, mesh=mesh,
        out_type=[
            jax.ShapeDtypeStruct((T, D), jnp.float32),
            jax.ShapeDtypeStruct((T, D), jnp.float32),
        ],
        scratch_types=[
            pltpu.VMEM((ch,), jnp.int32),
            pltpu.VMEM((ch,), jnp.int32),
            pltpu.VMEM((ch, D), jnp.float32),
            pltpu.VMEM((ch, D), jnp.float32),
            pltpu.SemaphoreType.DMA,
        ],
    )
    def combine(ys, pos1, pos2, y1_out, y2_out, idx1, idx2, r1, r2, sem):
        wid = lax.axis_index("s") * nc + lax.axis_index("c")
        base = wid * per_w

        def step(c, carry):
            off = base + c * ch
            pltpu.sync_copy(pos1.at[pl.ds(off, ch)], idx1)
            pltpu.sync_copy(pos2.at[pl.ds(off, ch)], idx2)
            g1 = pltpu.async_copy(ys.at[idx1], r1, sem)
            g2 = pltpu.async_copy(ys.at[idx2], r2, sem)
            g1.wait()
            g2.wait()
            pltpu.sync_copy(r1, y1_out.at[pl.ds(off, ch)])
            pltpu.sync_copy(r2, y2_out.at[pl.ds(off, ch)])
            return carry

        lax.fori_loop(0, per_w // ch, step, 0)

    return combine


def _wsum_body(y1_ref, y2_ref, p1_ref, p2_ref, out_ref):
    out_ref[...] = p1_ref[...] * y1_ref[...] + p2_ref[...] * y2_ref[...]


def _make_wsum(T, D):
    bt = 512
    return pl.pallas_call(
        _wsum_body,
        grid=(T // bt,),
        in_specs=[
            pl.BlockSpec((bt, D), lambda i: (i, 0)),
            pl.BlockSpec((bt, D), lambda i: (i, 0)),
            pl.BlockSpec((bt, 1), lambda i: (i, 0)),
            pl.BlockSpec((bt, 1), lambda i: (i, 0)),
        ],
        out_specs=pl.BlockSpec((bt, D), lambda i: (i, 0)),
        out_shape=jax.ShapeDtypeStruct((T, D), jnp.float32),
    )


def kernel(x, router_w, w1, w2, w3):
    B, S, D = x.shape
    T = B * S
    H = w1.shape[1]
    xf = x.reshape(T, D)
    rw_pad = jnp.zeros((128, D), jnp.float32).at[:E].set(router_w)

    aux, pos1, pos2, p1, p2, sinfo = _make_router(T, D)(xf, rw_pad)
    sblk = sinfo[0, :NB]
    snu = sinfo[0, 16:17]
    pos1f = pos1.reshape(T)
    pos2f = pos2.reshape(T)

    xs = _make_dispatch(T, D)(xf, pos1f, pos2f)
    ys = _make_ffn(D, H)(sblk, snu, xs, w1, w3, w2)
    y1, y2 = _make_combine(T, D)(ys, pos1f, pos2f)
    out = _make_wsum(T, D)(y1, y2, p1, p2)
    return out.reshape(B, S, D), aux[0, 0]


# speedup vs baseline: 1.2851x; 1.0475x over previous
import functools

import jax
import jax.numpy as jnp
from jax import lax
from jax.experimental import pallas as pl
from jax.experimental.pallas import tpu as pltpu
from jax.experimental.pallas import tpu_sc as plsc

E = 8
AUX_COEF = 0.01
NEG = -1e30
BM = 768
NB = 12
HT = 1024
TP = NB * BM


def _shift_down(a, d):
    pad = jnp.zeros((d, a.shape[1]), a.dtype)
    return jnp.concatenate([pad, a[:-d]], axis=0)


def _shift_right(a, d):
    pad = jnp.zeros((a.shape[0], d), a.dtype)
    return jnp.concatenate([pad, a[:, :-d]], axis=1)


def _cumsum_rows(a):
    d = 1
    while d < a.shape[0]:
        a = a + _shift_down(a, d)
        d *= 2
    return a


def _cumsum_lanes(a):
    d = 1
    while d < a.shape[1]:
        a = a + _shift_right(a, d)
        d *= 2
    return a


def _router_body(x_ref, rw_ref, aux_ref, pos1_ref, pos2_ref,
                 p1_ref, p2_ref, sinfo_ref):
    x = x_ref[...]
    rw = rw_ref[...]
    logits = jax.lax.dot_general(
        x, rw, (((1,), (1,)), ((), ())),
        preferred_element_type=jnp.float32)
    T = logits.shape[0]
    col = jax.lax.broadcasted_iota(jnp.int32, logits.shape, 1)
    valid = col < E
    logits = jnp.where(valid, logits, NEG)
    m = jnp.max(logits, axis=1, keepdims=True)
    ex = jnp.where(valid, jnp.exp(logits - m), 0.0)
    probs = ex / jnp.sum(ex, axis=1, keepdims=True)
    m1 = jnp.max(logits, axis=1, keepdims=True)
    am1 = jnp.min(jnp.where(logits == m1, col, 9999), axis=1, keepdims=True)
    p1 = jnp.max(probs, axis=1, keepdims=True)
    logits2 = jnp.where(col == am1, NEG, logits)
    m2 = jnp.max(logits2, axis=1, keepdims=True)
    am2 = jnp.min(jnp.where(logits2 == m2, col, 9999), axis=1, keepdims=True)
    p2 = jnp.max(jnp.where(col == am1, -1.0, probs), axis=1, keepdims=True)
    one1 = (col == am1).astype(jnp.float32)
    dens = jnp.sum(one1, axis=0, keepdims=True) / T
    rpm = jnp.sum(probs, axis=0, keepdims=True) / T
    aux_ref[...] = AUX_COEF * E * jnp.sum(dens * rpm, keepdims=True)
    p1_ref[...] = p1
    p2_ref[...] = p2
    oh1 = (col == am1).astype(jnp.int32)
    oh2 = (col == am2).astype(jnp.int32)
    cum1 = _cumsum_rows(oh1)
    cum2 = _cumsum_rows(oh2)
    cnt1 = cum1[-1:, :]
    cnt2 = cum2[-1:, :]
    counts = cnt1 + cnt2
    nblk = (counts + (BM - 1)) // BM
    cumblk = _cumsum_lanes(nblk)
    excl = cumblk - nblk
    nu = jnp.sum(nblk, axis=1, keepdims=True)
    start_pad = excl * BM
    pos1 = jnp.sum(oh1 * (start_pad + cum1 - 1), axis=1, keepdims=True)
    pos2 = jnp.sum(oh2 * (start_pad + cnt1 + cum2 - 1), axis=1, keepdims=True)
    pos1_ref[...] = pos1
    pos2_ref[...] = pos2
    biota = jax.lax.broadcasted_iota(jnp.int32, (1, 128), 1)
    mk = jnp.zeros((1, 128), jnp.int32)
    prev = jnp.zeros((1, 1), jnp.int32)
    for e in range(E):
        has = nblk[:, e:e + 1] > 0
        delta = jnp.where(has, e - prev, 0)
        mk = mk + jnp.where(biota == excl[:, e:e + 1], delta, 0)
        prev = jnp.where(has, e, prev)
    sblk = _cumsum_lanes(mk)
    sinfo_ref[...] = jnp.where(biota == 16, jnp.broadcast_to(nu, (1, 128)), sblk)


def _make_router(T, D):
    return pl.pallas_call(
        _router_body,
        out_shape=[
            jax.ShapeDtypeStruct((1, 1), jnp.float32),
            jax.ShapeDtypeStruct((T, 1), jnp.int32),
            jax.ShapeDtypeStruct((T, 1), jnp.int32),
            jax.ShapeDtypeStruct((T, 1), jnp.float32),
            jax.ShapeDtypeStruct((T, 1), jnp.float32),
            jax.ShapeDtypeStruct((1, 128), jnp.int32),
        ],
    )


def _make_dispatch(T, D):
    info = plsc.get_sparse_core_info()
    nw = info.num_cores * info.num_subcores
    nc = info.num_cores
    ch = T // nw
    mesh = plsc.VectorSubcoreMesh(core_axis_name="c", subcore_axis_name="s")

    @functools.partial(
        pl.kernel, mesh=mesh,
        out_type=jax.ShapeDtypeStruct((TP, D), jnp.float32),
        scratch_types=[
            pltpu.VMEM((ch, D), jnp.float32),
            pltpu.VMEM((ch,), jnp.int32),
            pltpu.VMEM((ch,), jnp.int32),
            pltpu.SemaphoreType.DMA,
        ],
    )
    def dispatch(xf, pos1, pos2, xs_out, xb, idx1, idx2, sem):
        wid = lax.axis_index("s") * nc + lax.axis_index("c")
        base = wid * ch
        pltpu.sync_copy(xf.at[pl.ds(base, ch)], xb)
        pltpu.sync_copy(pos1.at[pl.ds(base, ch)], idx1)
        pltpu.sync_copy(pos2.at[pl.ds(base, ch)], idx2)
        c1 = pltpu.async_copy(xb, xs_out.at[idx1], sem)
        c2 = pltpu.async_copy(xb, xs_out.at[idx2], sem)
        c1.wait()
        c2.wait()

    return dispatch


def _ffn_body(sblk_ref, snu_ref, xs_ref, w1_ref, w3_ref, w2_ref, ys_ref):
    b = pl.program_id(0)
    h = pl.program_id(1)

    @pl.when(b < snu_ref[0])
    def _():
        x = xs_ref[...]
        w1 = w1_ref[0]
        w3 = w3_ref[0]
        w2 = w2_ref[0]
        hh = jax.lax.dot_general(x, w1, (((1,), (1,)), ((), ())),
                                 preferred_element_type=jnp.float32)
        g = hh * jax.nn.sigmoid(hh)
        u = jax.lax.dot_general(x, w3, (((1,), (1,)), ((), ())),
                                preferred_element_type=jnp.float32)
        act = g * u
        y = jax.lax.dot_general(act, w2, (((1,), (1,)), ((), ())),
                                preferred_element_type=jnp.float32)

        @pl.when(h == 0)
        def _():
            ys_ref[...] = y

        @pl.when(h != 0)
        def _():
            ys_ref[...] += y


def _make_ffn(D, H):
    nh = H // HT

    def xs_idx(b, h, sblk, snu):
        return (jnp.minimum(b, snu[0] - 1), 0)

    def w13_idx(b, h, sblk, snu):
        return (sblk[b], jnp.where(b < snu[0], h, nh - 1), 0)

    def w2_idx(b, h, sblk, snu):
        return (sblk[b], 0, jnp.where(b < snu[0], h, nh - 1))

    grid_spec = pltpu.PrefetchScalarGridSpec(
        num_scalar_prefetch=2,
        grid=(NB, nh),
        in_specs=[
            pl.BlockSpec((BM, D), xs_idx),
            pl.BlockSpec((1, HT, D), w13_idx),
            pl.BlockSpec((1, HT, D), w13_idx),
            pl.BlockSpec((1, D, HT), w2_idx),
        ],
        out_specs=pl.BlockSpec((BM, D), xs_idx),
    )
    return pl.pallas_call(
        _ffn_body,
        grid_spec=grid_spec,
        out_shape=jax.ShapeDtypeStruct((TP, D), jnp.float32),
        compiler_params=pltpu.CompilerParams(
            dimension_semantics=("arbitrary", "arbitrary")),
    )


def _make_combine(T, D):
    info = plsc.get_sparse_core_info()
    nw = info.num_cores * info.num_subcores
    nc = info.num_cores
    per_w = T // nw
    ch = 16

    mesh = plsc.VectorSubcoreMesh(core_axis_name="c", subcore_axis_name="s")

    @functools.partial(
        pl.kernel, mesh=mesh,
        out_type=[
            jax.ShapeDtypeStruct((T, D), jnp.float32),
            jax.ShapeDtypeStruct((T, D), jnp.float32),
        ],
        scratch_types=[
            pltpu.VMEM((ch,), jnp.int32),
            pltpu.VMEM((ch,), jnp.int32),
            pltpu.VMEM((ch, D), jnp.float32),
            pltpu.VMEM((ch, D), jnp.float32),
            pltpu.SemaphoreType.DMA,
        ],
    )
    def combine(ys, pos1, pos2, y1_out, y2_out, idx1, idx2, r1, r2, sem):
        wid = lax.axis_index("s") * nc + lax.axis_index("c")
        base = wid * per_w

        def step(c, carry):
            off = base + c * ch
            pltpu.sync_copy(pos1.at[pl.ds(off, ch)], idx1)
            pltpu.sync_copy(pos2.at[pl.ds(off, ch)], idx2)
            g1 = pltpu.async_copy(ys.at[idx1], r1, sem)
            g2 = pltpu.async_copy(ys.at[idx2], r2, sem)
            g1.wait()
            g2.wait()
            pltpu.sync_copy(r1, y1_out.at[pl.ds(off, ch)])
            pltpu.sync_copy(r2, y2_out.at[pl.ds(off, ch)])
            return carry

        lax.fori_loop(0, per_w // ch, step, 0)

    return combine


def _wsum_body(y1_ref, y2_ref, p1_ref, p2_ref, out_ref):
    out_ref[...] = p1_ref[...] * y1_ref[...] + p2_ref[...] * y2_ref[...]


def _make_wsum(T, D):
    bt = 512
    return pl.pallas_call(
        _wsum_body,
        grid=(T // bt,),
        in_specs=[
            pl.BlockSpec((bt, D), lambda i: (i, 0)),
            pl.BlockSpec((bt, D), lambda i: (i, 0)),
            pl.BlockSpec((bt, 1), lambda i: (i, 0)),
            pl.BlockSpec((bt, 1), lambda i: (i, 0)),
        ],
        out_specs=pl.BlockSpec((bt, D), lambda i: (i, 0)),
        out_shape=jax.ShapeDtypeStruct((T, D), jnp.float32),
    )


def kernel(x, router_w, w1, w2, w3):
    B, S, D = x.shape
    T = B * S
    H = w1.shape[1]
    xf = x.reshape(T, D)
    rw_pad = jnp.zeros((128, D), jnp.float32).at[:E].set(router_w)

    aux, pos1, pos2, p1, p2, sinfo = _make_router(T, D)(xf, rw_pad)
    sblk = sinfo[0, :NB]
    snu = sinfo[0, 16:17]
    pos1f = pos1.reshape(T)
    pos2f = pos2.reshape(T)

    xs = _make_dispatch(T, D)(xf, pos1f, pos2f)
    ys = _make_ffn(D, H)(sblk, snu, xs, w1, w3, w2)
    y1, y2 = _make_combine(T, D)(ys, pos1f, pos2f)
    out = _make_wsum(T, D)(y1, y2, p1, p2)
    return out.reshape(B, S, D), aux[0, 0]


# speedup vs baseline: 1.4002x; 1.0895x over previous
import functools

import jax
import jax.numpy as jnp
from jax import lax
from jax.experimental import pallas as pl
from jax.experimental.pallas import tpu as pltpu
from jax.experimental.pallas import tpu_sc as plsc

E = 8
AUX_COEF = 0.01
NEG = -1e30
BM = 640
NB = 14
HT = 1024
TP = NB * BM


def _shift_down(a, d):
    pad = jnp.zeros((d, a.shape[1]), a.dtype)
    return jnp.concatenate([pad, a[:-d]], axis=0)


def _shift_right(a, d):
    pad = jnp.zeros((a.shape[0], d), a.dtype)
    return jnp.concatenate([pad, a[:, :-d]], axis=1)


def _cumsum_rows(a):
    d = 1
    while d < a.shape[0]:
        a = a + _shift_down(a, d)
        d *= 2
    return a


def _cumsum_lanes(a):
    d = 1
    while d < a.shape[1]:
        a = a + _shift_right(a, d)
        d *= 2
    return a


def _router_body(x_ref, rw_ref, aux_ref, pos1_ref, pos2_ref,
                 p1_ref, p2_ref, sinfo_ref):
    x = x_ref[...]
    rw = rw_ref[...]
    logits = jax.lax.dot_general(
        x, rw, (((1,), (1,)), ((), ())),
        preferred_element_type=jnp.float32)
    T = logits.shape[0]
    col = jax.lax.broadcasted_iota(jnp.int32, logits.shape, 1)
    valid = col < E
    logits = jnp.where(valid, logits, NEG)
    m = jnp.max(logits, axis=1, keepdims=True)
    ex = jnp.where(valid, jnp.exp(logits - m), 0.0)
    probs = ex / jnp.sum(ex, axis=1, keepdims=True)
    m1 = jnp.max(logits, axis=1, keepdims=True)
    am1 = jnp.min(jnp.where(logits == m1, col, 9999), axis=1, keepdims=True)
    p1 = jnp.max(probs, axis=1, keepdims=True)
    logits2 = jnp.where(col == am1, NEG, logits)
    m2 = jnp.max(logits2, axis=1, keepdims=True)
    am2 = jnp.min(jnp.where(logits2 == m2, col, 9999), axis=1, keepdims=True)
    p2 = jnp.max(jnp.where(col == am1, -1.0, probs), axis=1, keepdims=True)
    one1 = (col == am1).astype(jnp.float32)
    dens = jnp.sum(one1, axis=0, keepdims=True) / T
    rpm = jnp.sum(probs, axis=0, keepdims=True) / T
    aux_ref[...] = AUX_COEF * E * jnp.sum(dens * rpm, keepdims=True)
    p1_ref[...] = p1
    p2_ref[...] = p2
    oh1 = (col == am1).astype(jnp.int32)
    oh2 = (col == am2).astype(jnp.int32)
    cum1 = _cumsum_rows(oh1)
    cum2 = _cumsum_rows(oh2)
    cnt1 = cum1[-1:, :]
    cnt2 = cum2[-1:, :]
    counts = cnt1 + cnt2
    nblk = (counts + (BM - 1)) // BM
    cumblk = _cumsum_lanes(nblk)
    excl = cumblk - nblk
    nu = jnp.sum(nblk, axis=1, keepdims=True)
    start_pad = excl * BM
    pos1 = jnp.sum(oh1 * (start_pad + cum1 - 1), axis=1, keepdims=True)
    pos2 = jnp.sum(oh2 * (start_pad + cnt1 + cum2 - 1), axis=1, keepdims=True)
    pos1_ref[...] = pos1
    pos2_ref[...] = pos2
    biota = jax.lax.broadcasted_iota(jnp.int32, (1, 128), 1)
    mk = jnp.zeros((1, 128), jnp.int32)
    prev = jnp.zeros((1, 1), jnp.int32)
    for e in range(E):
        has = nblk[:, e:e + 1] > 0
        delta = jnp.where(has, e - prev, 0)
        mk = mk + jnp.where(biota == excl[:, e:e + 1], delta, 0)
        prev = jnp.where(has, e, prev)
    sblk = _cumsum_lanes(mk)
    sinfo_ref[...] = jnp.where(biota == 16, jnp.broadcast_to(nu, (1, 128)), sblk)


def _make_router(T, D):
    return pl.pallas_call(
        _router_body,
        out_shape=[
            jax.ShapeDtypeStruct((1, 1), jnp.float32),
            jax.ShapeDtypeStruct((T, 1), jnp.int32),
            jax.ShapeDtypeStruct((T, 1), jnp.int32),
            jax.ShapeDtypeStruct((T, 1), jnp.float32),
            jax.ShapeDtypeStruct((T, 1), jnp.float32),
            jax.ShapeDtypeStruct((1, 128), jnp.int32),
        ],
    )


def _make_dispatch(T, D):
    info = plsc.get_sparse_core_info()
    nw = info.num_cores * info.num_subcores
    nc = info.num_cores
    ch = T // nw
    mesh = plsc.VectorSubcoreMesh(core_axis_name="c", subcore_axis_name="s")

    @functools.partial(
        pl.kernel, mesh=mesh,
        out_type=jax.ShapeDtypeStruct((TP, D), jnp.float32),
        scratch_types=[
            pltpu.VMEM((ch, D), jnp.float32),
            pltpu.VMEM((ch,), jnp.int32),
            pltpu.VMEM((ch,), jnp.int32),
            pltpu.SemaphoreType.DMA,
        ],
    )
    def dispatch(xf, pos1, pos2, xs_out, xb, idx1, idx2, sem):
        wid = lax.axis_index("s") * nc + lax.axis_index("c")
        base = wid * ch
        pltpu.sync_copy(xf.at[pl.ds(base, ch)], xb)
        pltpu.sync_copy(pos1.at[pl.ds(base, ch)], idx1)
        pltpu.sync_copy(pos2.at[pl.ds(base, ch)], idx2)
        c1 = pltpu.async_copy(xb, xs_out.at[idx1], sem)
        c2 = pltpu.async_copy(xb, xs_out.at[idx2], sem)
        c1.wait()
        c2.wait()

    return dispatch


def _ffn_body(sblk_ref, snu_ref, xs_ref, w1_ref, w3_ref, w2_ref, ys_ref):
    b = pl.program_id(0)
    h = pl.program_id(1)

    @pl.when(b < snu_ref[0])
    def _():
        x = xs_ref[...]
        w1 = w1_ref[0]
        w3 = w3_ref[0]
        w2 = w2_ref[0]
        hh = jax.lax.dot_general(x, w1, (((1,), (1,)), ((), ())),
                                 preferred_element_type=jnp.float32)
        g = hh * jax.nn.sigmoid(hh)
        u = jax.lax.dot_general(x, w3, (((1,), (1,)), ((), ())),
                                preferred_element_type=jnp.float32)
        act = g * u
        y = jax.lax.dot_general(act, w2, (((1,), (1,)), ((), ())),
                                preferred_element_type=jnp.float32)

        @pl.when(h == 0)
        def _():
            ys_ref[...] = y

        @pl.when(h != 0)
        def _():
            ys_ref[...] += y


def _make_ffn(D, H):
    nh = H // HT

    def xs_idx(b, h, sblk, snu):
        return (jnp.minimum(b, snu[0] - 1), 0)

    def w13_idx(b, h, sblk, snu):
        return (sblk[b], jnp.where(b < snu[0], h, nh - 1), 0)

    def w2_idx(b, h, sblk, snu):
        return (sblk[b], 0, jnp.where(b < snu[0], h, nh - 1))

    grid_spec = pltpu.PrefetchScalarGridSpec(
        num_scalar_prefetch=2,
        grid=(NB, nh),
        in_specs=[
            pl.BlockSpec((BM, D), xs_idx),
            pl.BlockSpec((1, HT, D), w13_idx),
            pl.BlockSpec((1, HT, D), w13_idx),
            pl.BlockSpec((1, D, HT), w2_idx),
        ],
        out_specs=pl.BlockSpec((BM, D), xs_idx),
    )
    return pl.pallas_call(
        _ffn_body,
        grid_spec=grid_spec,
        out_shape=jax.ShapeDtypeStruct((TP, D), jnp.float32),
        compiler_params=pltpu.CompilerParams(
            dimension_semantics=("arbitrary", "arbitrary")),
    )


def _make_combine(T, D):
    info = plsc.get_sparse_core_info()
    nw = info.num_cores * info.num_subcores
    nc = info.num_cores
    per_w = T // nw
    ch = 16

    mesh = plsc.VectorSubcoreMesh(core_axis_name="c", subcore_axis_name="s")

    @functools.partial(
        pl.kernel, mesh=mesh,
        out_type=[
            jax.ShapeDtypeStruct((T, D), jnp.float32),
            jax.ShapeDtypeStruct((T, D), jnp.float32),
        ],
        scratch_types=[
            pltpu.VMEM((ch,), jnp.int32),
            pltpu.VMEM((ch,), jnp.int32),
            pltpu.VMEM((ch, D), jnp.float32),
            pltpu.VMEM((ch, D), jnp.float32),
            pltpu.SemaphoreType.DMA,
        ],
    )
    def combine(ys, pos1, pos2, y1_out, y2_out, idx1, idx2, r1, r2, sem):
        wid = lax.axis_index("s") * nc + lax.axis_index("c")
        base = wid * per_w

        def step(c, carry):
            off = base + c * ch
            pltpu.sync_copy(pos1.at[pl.ds(off, ch)], idx1)
            pltpu.sync_copy(pos2.at[pl.ds(off, ch)], idx2)
            g1 = pltpu.async_copy(ys.at[idx1], r1, sem)
            g2 = pltpu.async_copy(ys.at[idx2], r2, sem)
            g1.wait()
            g2.wait()
            pltpu.sync_copy(r1, y1_out.at[pl.ds(off, ch)])
            pltpu.sync_copy(r2, y2_out.at[pl.ds(off, ch)])
            return carry

        lax.fori_loop(0, per_w // ch, step, 0)

    return combine


def _wsum_body(y1_ref, y2_ref, p1_ref, p2_ref, out_ref):
    out_ref[...] = p1_ref[...] * y1_ref[...] + p2_ref[...] * y2_ref[...]


def _make_wsum(T, D):
    bt = 512
    return pl.pallas_call(
        _wsum_body,
        grid=(T // bt,),
        in_specs=[
            pl.BlockSpec((bt, D), lambda i: (i, 0)),
            pl.BlockSpec((bt, D), lambda i: (i, 0)),
            pl.BlockSpec((bt, 1), lambda i: (i, 0)),
            pl.BlockSpec((bt, 1), lambda i: (i, 0)),
        ],
        out_specs=pl.BlockSpec((bt, D), lambda i: (i, 0)),
        out_shape=jax.ShapeDtypeStruct((T, D), jnp.float32),
    )


def kernel(x, router_w, w1, w2, w3):
    B, S, D = x.shape
    T = B * S
    H = w1.shape[1]
    xf = x.reshape(T, D)
    rw_pad = jnp.zeros((128, D), jnp.float32).at[:E].set(router_w)

    aux, pos1, pos2, p1, p2, sinfo = _make_router(T, D)(xf, rw_pad)
    sblk = sinfo[0, :NB]
    snu = sinfo[0, 16:17]
    pos1f = pos1.reshape(T)
    pos2f = pos2.reshape(T)

    xs = _make_dispatch(T, D)(xf, pos1f, pos2f)
    ys = _make_ffn(D, H)(sblk, snu, xs, w1, w3, w2)
    y1, y2 = _make_combine(T, D)(ys, pos1f, pos2f)
    out = _make_wsum(T, D)(y1, y2, p1, p2)
    return out.reshape(B, S, D), aux[0, 0]


# speedup vs baseline: 1.4245x; 1.0174x over previous
import functools

import jax
import jax.numpy as jnp
from jax import lax
from jax.experimental import pallas as pl
from jax.experimental.pallas import tpu as pltpu
from jax.experimental.pallas import tpu_sc as plsc

E = 8
AUX_COEF = 0.01
NEG = -1e30
BM = 640
NB = 14
HT = 1024
TP = NB * BM


def _shift_down(a, d):
    pad = jnp.zeros((d, a.shape[1]), a.dtype)
    return jnp.concatenate([pad, a[:-d]], axis=0)


def _shift_right(a, d):
    pad = jnp.zeros((a.shape[0], d), a.dtype)
    return jnp.concatenate([pad, a[:, :-d]], axis=1)


def _cumsum_rows(a):
    d = 1
    while d < a.shape[0]:
        a = a + _shift_down(a, d)
        d *= 2
    return a


def _cumsum_lanes(a):
    d = 1
    while d < a.shape[1]:
        a = a + _shift_right(a, d)
        d *= 2
    return a


def _router_body(x_ref, rw_ref, aux_ref, pos1_ref, pos2_ref,
                 p1_ref, p2_ref, sinfo_ref):
    x = x_ref[...]
    rw = rw_ref[...]
    logits = jax.lax.dot_general(
        x, rw, (((1,), (1,)), ((), ())),
        preferred_element_type=jnp.float32)
    T = logits.shape[0]
    col = jax.lax.broadcasted_iota(jnp.int32, logits.shape, 1)
    valid = col < E
    logits = jnp.where(valid, logits, NEG)
    m = jnp.max(logits, axis=1, keepdims=True)
    ex = jnp.where(valid, jnp.exp(logits - m), 0.0)
    probs = ex / jnp.sum(ex, axis=1, keepdims=True)
    m1 = jnp.max(logits, axis=1, keepdims=True)
    am1 = jnp.min(jnp.where(logits == m1, col, 9999), axis=1, keepdims=True)
    p1 = jnp.max(probs, axis=1, keepdims=True)
    logits2 = jnp.where(col == am1, NEG, logits)
    m2 = jnp.max(logits2, axis=1, keepdims=True)
    am2 = jnp.min(jnp.where(logits2 == m2, col, 9999), axis=1, keepdims=True)
    p2 = jnp.max(jnp.where(col == am1, -1.0, probs), axis=1, keepdims=True)
    one1 = (col == am1).astype(jnp.float32)
    dens = jnp.sum(one1, axis=0, keepdims=True) / T
    rpm = jnp.sum(probs, axis=0, keepdims=True) / T
    aux_ref[...] = AUX_COEF * E * jnp.sum(dens * rpm, keepdims=True)
    p1_ref[...] = p1
    p2_ref[...] = p2
    oh1 = (col == am1).astype(jnp.int32)
    oh2 = (col == am2).astype(jnp.int32)
    cum1 = _cumsum_rows(oh1)
    cum2 = _cumsum_rows(oh2)
    cnt1 = cum1[-1:, :]
    cnt2 = cum2[-1:, :]
    counts = cnt1 + cnt2
    nblk = (counts + (BM - 1)) // BM
    cumblk = _cumsum_lanes(nblk)
    excl = cumblk - nblk
    nu = jnp.sum(nblk, axis=1, keepdims=True)
    start_pad = excl * BM
    pos1 = jnp.sum(oh1 * (start_pad + cum1 - 1), axis=1, keepdims=True)
    pos2 = jnp.sum(oh2 * (start_pad + cnt1 + cum2 - 1), axis=1, keepdims=True)
    pos1_ref[...] = pos1
    pos2_ref[...] = pos2
    biota = jax.lax.broadcasted_iota(jnp.int32, (1, 128), 1)
    mk = jnp.zeros((1, 128), jnp.int32)
    prev = jnp.zeros((1, 1), jnp.int32)
    for e in range(E):
        has = nblk[:, e:e + 1] > 0
        delta = jnp.where(has, e - prev, 0)
        mk = mk + jnp.where(biota == excl[:, e:e + 1], delta, 0)
        prev = jnp.where(has, e, prev)
    sblk = _cumsum_lanes(mk)
    sinfo_ref[...] = jnp.where(biota == 16, jnp.broadcast_to(nu, (1, 128)), sblk)


def _make_router(T, D):
    return pl.pallas_call(
        _router_body,
        out_shape=[
            jax.ShapeDtypeStruct((1, 1), jnp.float32),
            jax.ShapeDtypeStruct((T, 1), jnp.int32),
            jax.ShapeDtypeStruct((T, 1), jnp.int32),
            jax.ShapeDtypeStruct((T, 1), jnp.float32),
            jax.ShapeDtypeStruct((T, 1), jnp.float32),
            jax.ShapeDtypeStruct((1, 128), jnp.int32),
        ],
    )


def _make_dispatch(T, D):
    info = plsc.get_sparse_core_info()
    nw = info.num_cores * info.num_subcores
    nc = info.num_cores
    ch = T // nw
    mesh = plsc.VectorSubcoreMesh(core_axis_name="c", subcore_axis_name="s")

    @functools.partial(
        pl.kernel, mesh=mesh,
        out_type=jax.ShapeDtypeStruct((TP, D), jnp.float32),
        scratch_types=[
            pltpu.VMEM((ch, D), jnp.float32),
            pltpu.VMEM((ch,), jnp.int32),
            pltpu.VMEM((ch,), jnp.int32),
            pltpu.SemaphoreType.DMA,
        ],
    )
    def dispatch(xf, pos1, pos2, xs_out, xb, idx1, idx2, sem):
        wid = lax.axis_index("s") * nc + lax.axis_index("c")
        base = wid * ch
        pltpu.sync_copy(xf.at[pl.ds(base, ch)], xb)
        pltpu.sync_copy(pos1.at[pl.ds(base, ch)], idx1)
        pltpu.sync_copy(pos2.at[pl.ds(base, ch)], idx2)
        c1 = pltpu.async_copy(xb, xs_out.at[idx1], sem)
        c2 = pltpu.async_copy(xb, xs_out.at[idx2], sem)
        c1.wait()
        c2.wait()

    return dispatch


def _ffn_body(sblk_ref, snu_ref, xs_ref, w1_ref, w3_ref, w2_ref, ys_ref):
    b = pl.program_id(0)
    h = pl.program_id(1)

    @pl.when(b < snu_ref[0])
    def _():
        x = xs_ref[...]
        w1 = w1_ref[0]
        w3 = w3_ref[0]
        w2 = w2_ref[0]
        hh = jax.lax.dot_general(x, w1, (((1,), (1,)), ((), ())),
                                 preferred_element_type=jnp.float32)
        g = hh * jax.nn.sigmoid(hh)
        u = jax.lax.dot_general(x, w3, (((1,), (1,)), ((), ())),
                                preferred_element_type=jnp.float32)
        act = g * u
        y = jax.lax.dot_general(act, w2, (((1,), (1,)), ((), ())),
                                preferred_element_type=jnp.float32)

        @pl.when(h == 0)
        def _():
            ys_ref[...] = y

        @pl.when(h != 0)
        def _():
            ys_ref[...] += y


def _make_ffn(D, H):
    nh = H // HT

    def xs_idx(b, h, sblk, snu):
        return (jnp.minimum(b, snu[0] - 1), 0)

    def w13_idx(b, h, sblk, snu):
        return (sblk[b], jnp.where(b < snu[0], h, nh - 1), 0)

    def w2_idx(b, h, sblk, snu):
        return (sblk[b], 0, jnp.where(b < snu[0], h, nh - 1))

    grid_spec = pltpu.PrefetchScalarGridSpec(
        num_scalar_prefetch=2,
        grid=(NB, nh),
        in_specs=[
            pl.BlockSpec((BM, D), xs_idx),
            pl.BlockSpec((1, HT, D), w13_idx),
            pl.BlockSpec((1, HT, D), w13_idx),
            pl.BlockSpec((1, D, HT), w2_idx),
        ],
        out_specs=pl.BlockSpec((BM, D), xs_idx),
    )
    return pl.pallas_call(
        _ffn_body,
        grid_spec=grid_spec,
        out_shape=jax.ShapeDtypeStruct((TP, D), jnp.float32),
        compiler_params=pltpu.CompilerParams(
            dimension_semantics=("arbitrary", "arbitrary")),
    )


def _make_combine(T, D):
    info = plsc.get_sparse_core_info()
    nw = info.num_cores * info.num_subcores
    nc = info.num_cores
    per_w = T // nw
    ch = 32

    mesh = plsc.VectorSubcoreMesh(core_axis_name="c", subcore_axis_name="s")

    @functools.partial(
        pl.kernel, mesh=mesh,
        out_type=[
            jax.ShapeDtypeStruct((T, D), jnp.float32),
            jax.ShapeDtypeStruct((T, D), jnp.float32),
        ],
        scratch_types=[
            pltpu.VMEM((ch,), jnp.int32),
            pltpu.VMEM((ch,), jnp.int32),
            pltpu.VMEM((ch, D), jnp.float32),
            pltpu.VMEM((ch, D), jnp.float32),
            pltpu.SemaphoreType.DMA,
        ],
    )
    def combine(ys, pos1, pos2, y1_out, y2_out, idx1, idx2, r1, r2, sem):
        wid = lax.axis_index("s") * nc + lax.axis_index("c")
        base = wid * per_w

        def step(c, carry):
            off = base + c * ch
            pltpu.sync_copy(pos1.at[pl.ds(off, ch)], idx1)
            pltpu.sync_copy(pos2.at[pl.ds(off, ch)], idx2)
            g1 = pltpu.async_copy(ys.at[idx1], r1, sem)
            g2 = pltpu.async_copy(ys.at[idx2], r2, sem)
            g1.wait()
            g2.wait()
            pltpu.sync_copy(r1, y1_out.at[pl.ds(off, ch)])
            pltpu.sync_copy(r2, y2_out.at[pl.ds(off, ch)])
            return carry

        lax.fori_loop(0, per_w // ch, step, 0)

    return combine


def _wsum_body(y1_ref, y2_ref, p1_ref, p2_ref, out_ref):
    out_ref[...] = p1_ref[...] * y1_ref[...] + p2_ref[...] * y2_ref[...]


def _make_wsum(T, D):
    bt = 512
    return pl.pallas_call(
        _wsum_body,
        grid=(T // bt,),
        in_specs=[
            pl.BlockSpec((bt, D), lambda i: (i, 0)),
            pl.BlockSpec((bt, D), lambda i: (i, 0)),
            pl.BlockSpec((bt, 1), lambda i: (i, 0)),
            pl.BlockSpec((bt, 1), lambda i: (i, 0)),
        ],
        out_specs=pl.BlockSpec((bt, D), lambda i: (i, 0)),
        out_shape=jax.ShapeDtypeStruct((T, D), jnp.float32),
    )


def kernel(x, router_w, w1, w2, w3):
    B, S, D = x.shape
    T = B * S
    H = w1.shape[1]
    xf = x.reshape(T, D)
    rw_pad = jnp.zeros((128, D), jnp.float32).at[:E].set(router_w)

    aux, pos1, pos2, p1, p2, sinfo = _make_router(T, D)(xf, rw_pad)
    sblk = sinfo[0, :NB]
    snu = sinfo[0, 16:17]
    pos1f = pos1.reshape(T)
    pos2f = pos2.reshape(T)

    xs = _make_dispatch(T, D)(xf, pos1f, pos2f)
    ys = _make_ffn(D, H)(sblk, snu, xs, w1, w3, w2)
    y1, y2 = _make_combine(T, D)(ys, pos1f, pos2f)
    out = _make_wsum(T, D)(y1, y2, p1, p2)
    return out.reshape(B, S, D), aux[0, 0]
